# baseline (speedup 1.0000x reference)
import functools

import jax
import jax.numpy as jnp
from jax import lax
from jax.experimental import pallas as pl
from jax.experimental.pallas import tpu as pltpu
from jax.experimental.pallas import tpu_sc as plsc

BATCH = 16384
HIST = 50
EMB = 32
N = BATCH * HIST

NUM_CORES = 2
NUM_SUBCORES = 16
NW = NUM_CORES * NUM_SUBCORES
ROWS_W = BATCH // NW
RCHUNK = 16
CHUNK = RCHUNK * HIST
NCHUNK = ROWS_W // RCHUNK
NBUF = 3

_mesh = plsc.VectorSubcoreMesh(core_axis_name="c", subcore_axis_name="s")

_scratch = (
    [pltpu.VMEM((CHUNK,), jnp.int32) for _ in range(NBUF)]
    + [pltpu.VMEM((CHUNK, EMB), jnp.float32) for _ in range(NBUF)]
    + [pltpu.SemaphoreType.DMA for _ in range(3 * NBUF)]
)


@functools.partial(
    pl.kernel,
    mesh=_mesh,
    out_type=jax.ShapeDtypeStruct((N, EMB), jnp.float32),
    scratch_types=_scratch,
    compiler_params=pltpu.CompilerParams(use_tc_tiling_on_sc=False, needs_layout_passes=True),
)
def _gather_kernel(idx_hbm, table_hbm, out_hbm, *refs):
    idx_v = refs[0:NBUF]
    rows_v = refs[NBUF : 2 * NBUF]
    sems = refs[2 * NBUF :]
    si = sems[0:NBUF]
    sg = sems[NBUF : 2 * NBUF]
    so = sems[2 * NBUF :]

    wid = lax.axis_index("s") * NUM_CORES + lax.axis_index("c")
    rbase = wid * ROWS_W

    h_i = [None] * NCHUNK
    h_g = [None] * NCHUNK
    h_o = [None] * NCHUNK

    def stage_idx(i, b):
        return pltpu.async_copy(
            idx_hbm.at[pl.ds((rbase + i * RCHUNK) * HIST, CHUNK)],
            idx_v[b],
            si[b],
        )

    def stage_gather(i):
        b = i % NBUF
        h_i[i].wait()
        if i >= NBUF:
            for h in h_o[i - NBUF]:
                h.wait()
        h_g[i] = pltpu.async_copy(table_hbm.at[idx_v[b]], rows_v[b], sg[b])

    for i in range(min(NBUF, NCHUNK)):
        h_i[i] = stage_idx(i, i)
    stage_gather(0)

    for i in range(NCHUNK):
        b = i % NBUF
        if i + 1 < NCHUNK:
            stage_gather(i + 1)
        h_g[i].wait()
        if i + NBUF < NCHUNK:
            h_i[i + NBUF] = stage_idx(i + NBUF, b)
        h_o[i] = [
            pltpu.async_copy(
                rows_v[b],
                out_hbm.at[pl.ds((rbase + i * RCHUNK) * HIST, CHUNK)],
                so[b],
            )
        ]

    for i in range(max(0, NCHUNK - NBUF), NCHUNK):
        for h in h_o[i]:
            h.wait()


def kernel(x, table):
    out = _gather_kernel(x.reshape(-1).astype(jnp.int32), table)
    return out.reshape(BATCH, HIST, EMB)


# speedup vs baseline: 1.6218x; 1.6218x over previous
import functools

import jax
import jax.numpy as jnp
from jax import lax
from jax.experimental import pallas as pl
from jax.experimental.pallas import tpu as pltpu
from jax.experimental.pallas import tpu_sc as plsc

BATCH = 16384
HIST = 50
EMB = 32
N = BATCH * HIST

NUM_CORES = 2
NUM_SUBCORES = 16
NW = NUM_CORES * NUM_SUBCORES
ROWS_W = BATCH // NW
RCHUNK = 16
CHUNK = RCHUNK * HIST
NCHUNK = ROWS_W // RCHUNK
NBUF = 3

_mesh = plsc.VectorSubcoreMesh(core_axis_name="c", subcore_axis_name="s")

_scratch = (
    [pltpu.VMEM((CHUNK,), jnp.int32) for _ in range(NBUF)]
    + [pltpu.VMEM((CHUNK, EMB), jnp.float32) for _ in range(NBUF)]
    + [pltpu.SemaphoreType.DMA for _ in range(3 * NBUF)]
)


@functools.partial(
    pl.kernel,
    mesh=_mesh,
    out_type=jax.ShapeDtypeStruct((BATCH, HIST, EMB), jnp.float32),
    scratch_types=_scratch,
    compiler_params=pltpu.CompilerParams(use_tc_tiling_on_sc=False, needs_layout_passes=True),
)
def _gather_kernel(idx_hbm, table_hbm, out_hbm, *refs):
    idx_v = refs[0:NBUF]
    rows_v = refs[NBUF : 2 * NBUF]
    sems = refs[2 * NBUF :]
    si = sems[0:NBUF]
    sg = sems[NBUF : 2 * NBUF]
    so = sems[2 * NBUF :]

    wid = lax.axis_index("s") * NUM_CORES + lax.axis_index("c")
    rbase = wid * ROWS_W

    h_i = [None] * NCHUNK
    h_g = [None] * NCHUNK
    h_o = [None] * NCHUNK

    def stage_idx(i, b):
        return pltpu.async_copy(
            idx_hbm.at[pl.ds((rbase + i * RCHUNK) * HIST, CHUNK)],
            idx_v[b],
            si[b],
        )

    def stage_gather(i):
        b = i % NBUF
        h_i[i].wait()
        if i >= NBUF:
            for h in h_o[i - NBUF]:
                h.wait()
        h_g[i] = pltpu.async_copy(table_hbm.at[idx_v[b]], rows_v[b], sg[b])

    for i in range(min(NBUF, NCHUNK)):
        h_i[i] = stage_idx(i, i)
    stage_gather(0)

    for i in range(NCHUNK):
        b = i % NBUF
        if i + 1 < NCHUNK:
            stage_gather(i + 1)
        h_g[i].wait()
        if i + NBUF < NCHUNK:
            h_i[i + NBUF] = stage_idx(i + NBUF, b)
        h_o[i] = [
            pltpu.async_copy(
                rows_v[b].at[pl.ds(r * HIST, HIST)],
                out_hbm.at[rbase + i * RCHUNK + r],
                so[b],
            )
            for r in range(RCHUNK)
        ]

    for i in range(max(0, NCHUNK - NBUF), NCHUNK):
        for h in h_o[i]:
            h.wait()


def kernel(x, table):
    idx = jnp.maximum(x.reshape(-1).astype(jnp.int32), 0)
    tab = table + jnp.float32(0.0)
    return _gather_kernel(idx, tab) + jnp.float32(0.0)
